# baseline (speedup 1.0000x reference)
import jax
import jax.numpy as jnp
from jax.experimental import pallas as pl
from jax.experimental.pallas import tpu as pltpu

H = 16
C = 128
LOG2E = 1.4426950408889634


def _stage_a_kernel(cl_ref, q_ref, wq_ref, qh_ref, xs_ref, cnt_ref):
    nb = cl_ref.shape[0]
    nblk = q_ref.shape[1]
    x = q_ref[0].astype(jnp.bfloat16)
    qh_ref[0] = jax.lax.dot_general(
        x, wq_ref[...], (((1,), (1,)), ((), ())),
        preferred_element_type=jnp.float32).astype(jnp.bfloat16)

    iota = jax.lax.broadcasted_iota(jnp.int32, (C, nblk), 0)
    ohs = [(iota == cl_ref[r]).astype(jnp.bfloat16) for r in range(nb)]
    oh_all = jnp.concatenate(ohs, axis=0)
    ps = jax.lax.dot_general(
        oh_all, x, (((1,), (0,)), ((), ())),
        preferred_element_type=jnp.float32)
    pc = jnp.sum(oh_all.astype(jnp.float32), axis=1).reshape(1, nb * C)

    i = pl.program_id(1)

    @pl.when(i == 0)
    def _():
        xs_ref[0] = ps
        cnt_ref[0] = pc

    @pl.when(i != 0)
    def _():
        xs_ref[0] += ps
        cnt_ref[0] += pc


def _stage_b_kernel(xs_ref, cnt_ref, wkv_ref, cent_ref):
    d = xs_ref.shape[2]
    dh = d // H
    cents = jax.lax.dot_general(
        xs_ref[0].astype(jnp.bfloat16), wkv_ref[...], (((1,), (1,)), ((), ())),
        preferred_element_type=jnp.float32)
    cnt_col = cnt_ref[0].reshape(-1, 1)
    w_col = jnp.where(cnt_col > 0, 1.0 / cnt_col, 0.0)
    a_col = w_col * (LOG2E * jax.lax.rsqrt(jnp.float32(dh)))
    nrow = cents.shape[0]
    scale = jnp.concatenate(
        [jnp.broadcast_to(a_col, (nrow, d)),
         jnp.broadcast_to(w_col, (nrow, d))], axis=1)
    cent_ref[0] = (cents * scale).astype(jnp.bfloat16)


def _stage_c_kernel(qh_ref, cent_ref, cnt_ref, wp_ref, bp_ref, out_ref):
    nblk, d = qh_ref.shape[1], qh_ref.shape[2]
    nb = cnt_ref.shape[2] // C
    dh = d // H
    qh = qh_ref[0]
    lc = jnp.log2(cnt_ref[0])

    outs = []
    for h in range(H):
        r = h % nb
        rs = slice(r * C, (r + 1) * C)
        qh_h = qh[:, h * dh:(h + 1) * dh]
        kc = cent_ref[0, rs, h * dh:(h + 1) * dh]
        vc = cent_ref[0, rs, d + h * dh:d + (h + 1) * dh]
        t = jax.lax.dot_general(
            qh_h, kc, (((1,), (1,)), ((), ())),
            preferred_element_type=jnp.float32) + lc[:, rs]
        m = jnp.max(t, axis=1, keepdims=True)
        e = jnp.exp2(t - m)
        denom = jnp.sum(e, axis=1, keepdims=True)
        num = jax.lax.dot_general(
            e.astype(jnp.bfloat16), vc, (((1,), (0,)), ((), ())),
            preferred_element_type=jnp.float32)
        outs.append((num * (1.0 / denom)).astype(jnp.bfloat16))
    ao = jnp.concatenate(outs, axis=1)

    out_ref[0] = jax.lax.dot_general(
        ao, wp_ref[...], (((1,), (1,)), ((), ())),
        preferred_element_type=jnp.float32) + bp_ref[...]


def kernel(cluster, q, Wq, Wkv, Wp, bp):
    B, N, D = q.shape
    cl3 = cluster.reshape(B, 1, N)
    bp2 = bp.reshape(1, D)
    na = 4
    nc = 2
    blk_a = N // na
    blk_c = N // nc

    qh, xs, cnt = pl.pallas_call(
        _stage_a_kernel,
        grid=(B, na),
        in_specs=[
            pl.BlockSpec((B, 1, blk_a), lambda b, i: (0, 0, i)),
            pl.BlockSpec((1, blk_a, D), lambda b, i: (b, i, 0)),
            pl.BlockSpec((D, D), lambda b, i: (0, 0)),
        ],
        out_specs=[
            pl.BlockSpec((1, blk_a, D), lambda b, i: (b, i, 0)),
            pl.BlockSpec((1, B * C, D), lambda b, i: (b, 0, 0)),
            pl.BlockSpec((1, 1, B * C), lambda b, i: (b, 0, 0)),
        ],
        out_shape=[
            jax.ShapeDtypeStruct((B, N, D), jnp.bfloat16),
            jax.ShapeDtypeStruct((B, B * C, D), jnp.float32),
            jax.ShapeDtypeStruct((B, 1, B * C), jnp.float32),
        ],
    )(cl3, q, Wq.astype(jnp.bfloat16))

    cents = pl.pallas_call(
        _stage_b_kernel,
        grid=(B,),
        in_specs=[
            pl.BlockSpec((1, B * C, D), lambda b: (b, 0, 0)),
            pl.BlockSpec((1, 1, B * C), lambda b: (b, 0, 0)),
            pl.BlockSpec((2 * D, D), lambda b: (0, 0)),
        ],
        out_specs=pl.BlockSpec((1, B * C, 2 * D), lambda b: (b, 0, 0)),
        out_shape=jax.ShapeDtypeStruct((B, B * C, 2 * D), jnp.bfloat16),
    )(xs, cnt, Wkv.astype(jnp.bfloat16))

    out = pl.pallas_call(
        _stage_c_kernel,
        grid=(B, nc),
        in_specs=[
            pl.BlockSpec((1, blk_c, D), lambda b, i: (b, i, 0)),
            pl.BlockSpec((1, B * C, 2 * D), lambda b, i: (b, 0, 0)),
            pl.BlockSpec((1, 1, B * C), lambda b, i: (b, 0, 0)),
            pl.BlockSpec((D, D), lambda b, i: (0, 0)),
            pl.BlockSpec((1, D), lambda b, i: (0, 0)),
        ],
        out_specs=pl.BlockSpec((1, blk_c, D), lambda b, i: (b, i, 0)),
        out_shape=jax.ShapeDtypeStruct((B, N, D), jnp.float32),
    )(qh, cents, cnt, Wp.astype(jnp.bfloat16), bp2)

    return out


# speedup vs baseline: 1.1996x; 1.1996x over previous
import jax
import jax.numpy as jnp
from jax.experimental import pallas as pl
from jax.experimental.pallas import tpu as pltpu

H = 16
C = 128


def _mega_kernel(cl_ref, q_ref, wq_ref, wkv_ref, wp_ref, bp_ref, out_ref):
    n, d = q_ref.shape[1], q_ref.shape[2]
    nb = cl_ref.shape[0]
    dh = d // H
    x = q_ref[0]
    qh = jax.lax.dot_general(
        x, wq_ref[...], (((1,), (1,)), ((), ())),
        preferred_element_type=jnp.float32).astype(jnp.bfloat16)

    iota = jax.lax.broadcasted_iota(jnp.int32, (C, n), 0)
    log2e = 1.4426950408889634
    ohs = [(iota == cl_ref[r]).astype(jnp.bfloat16) for r in range(nb)]
    oh_all = jnp.concatenate(ohs, axis=0)
    cnt_col = jnp.sum(oh_all.astype(jnp.float32), axis=1, keepdims=True)
    xs = jax.lax.dot_general(
        oh_all, x, (((1,), (0,)), ((), ())),
        preferred_element_type=jnp.float32).astype(jnp.bfloat16)
    cents = jax.lax.dot_general(
        xs, wkv_ref[...], (((1,), (1,)), ((), ())),
        preferred_element_type=jnp.float32)

    w_col = jnp.where(cnt_col > 0, 1.0 / cnt_col, 0.0)
    a_col = w_col * (log2e * jax.lax.rsqrt(jnp.float32(dh)))
    lc_rows = [jnp.log2(cnt_col[r * C:(r + 1) * C]).reshape(1, C)
               for r in range(nb)]
    outs = []
    for h in range(H):
        r = h % nb
        rs = slice(r * C, (r + 1) * C)
        qh_h = qh[:, h * dh:(h + 1) * dh]
        kc = (cents[rs, h * dh:(h + 1) * dh]
              * a_col[rs]).astype(jnp.bfloat16)
        vc = (cents[rs, d + h * dh:d + (h + 1) * dh]
              * w_col[rs]).astype(jnp.bfloat16)
        t = jax.lax.dot_general(
            qh_h, kc, (((1,), (1,)), ((), ())),
            preferred_element_type=jnp.float32) + lc_rows[r]
        m = jnp.max(t, axis=1, keepdims=True)
        e = jnp.exp2(t - m)
        denom = jnp.sum(e, axis=1, keepdims=True)
        num = jax.lax.dot_general(
            e.astype(jnp.bfloat16), vc, (((1,), (0,)), ((), ())),
            preferred_element_type=jnp.float32)
        outs.append((num * (1.0 / denom)).astype(jnp.bfloat16))
    ao = jnp.concatenate(outs, axis=1)

    out_ref[0] = jax.lax.dot_general(
        ao, wp_ref[...], (((1,), (1,)), ((), ())),
        preferred_element_type=jnp.float32) + bp_ref[...]


def kernel(cluster, q, Wq, Wkv, Wp, bp):
    B, N, D = q.shape
    cl3 = cluster.reshape(B, 1, N)
    bp2 = bp.reshape(1, D)

    out = pl.pallas_call(
        _mega_kernel,
        grid=(B,),
        in_specs=[
            pl.BlockSpec((B, 1, N), lambda b: (0, 0, 0)),
            pl.BlockSpec((1, N, D), lambda b: (b, 0, 0)),
            pl.BlockSpec((D, D), lambda b: (0, 0)),
            pl.BlockSpec((2 * D, D), lambda b: (0, 0)),
            pl.BlockSpec((D, D), lambda b: (0, 0)),
            pl.BlockSpec((1, D), lambda b: (0, 0)),
        ],
        out_specs=pl.BlockSpec((1, N, D), lambda b: (b, 0, 0)),
        out_shape=jax.ShapeDtypeStruct((B, N, D), jnp.float32),
    )(cl3, q.astype(jnp.bfloat16), Wq.astype(jnp.bfloat16), Wkv.astype(jnp.bfloat16),
      Wp.astype(jnp.bfloat16), bp2)

    return out
